# baseline (speedup 1.0000x reference)
import functools
import jax
import jax.numpy as jnp
from jax import lax
from jax.experimental import pallas as pl
from jax.experimental.pallas import tpu as pltpu
from jax.experimental.pallas import tpu_sc as plsc

_L = 16
_NPAIR = 8
_NB = 4096
_RING = 3
_PF = 1


def _pe_add_kernel(P, B, PW):
    info = plsc.get_sparse_core_info()
    NC, NS = info.num_cores, info.num_subcores
    NW = NC * NS
    assert P % (NW * _L) == 0 and B % _NB == 0
    rows_per_w = P // NW
    assert rows_per_w % _L == 0 and rows_per_w % _NPAIR == 0
    assert _NPAIR <= 8 and 8 % _NPAIR == 0
    n_halves = B // _NB
    n_chunks = (rows_per_w // _NPAIR) * n_halves
    mesh = plsc.VectorSubcoreMesh(core_axis_name="c", subcore_axis_name="s")

    head = list(range(_RING - _PF))
    main_lo = len(head)
    main_hi = main_lo + ((n_chunks - _PF - main_lo) // _RING) * _RING
    tail = list(range(main_hi, n_chunks))

    @functools.partial(
        pl.kernel,
        out_type=jax.ShapeDtypeStruct((P, B), jnp.float32),
        mesh=mesh,
        scratch_types=[
            pltpu.VMEM((P // PW, PW), jnp.float32),
            pltpu.VMEM((rows_per_w // 8, 8 * _L), jnp.float32),
            [pltpu.VMEM((_NPAIR, _NB), jnp.float32)] * _RING,
            [pltpu.SemaphoreType.DMA] * _RING,
            [pltpu.SemaphoreType.DMA] * _RING,
        ],
    )
    def _k(x_hbm, pe_hbm, out_hbm, pe_v, pe_sv, bufs, isems, osems):
        wid = lax.axis_index("s") * NC + lax.axis_index("c")
        pltpu.sync_copy(pe_hbm, pe_v)
        base = wid * rows_per_w

        @pl.loop(0, rows_per_w // _L)
        def _mk_splat(k):
            p0 = base + k * _L
            pev = pe_v[lax.shift_right_logical(p0, 7),
                       pl.ds(lax.bitwise_and(p0, PW - 1), _L)]
            k2 = 2 * k
            for j in range(_L):
                pe_sv[k2 + (j >> 3), pl.ds((j & 7) * _L, _L)] = (
                    lax.broadcast_in_dim(pev[j], (_L,), ()))

        def window(c):
            lrow = (c // n_halves) * _NPAIR
            col0 = (c % n_halves) * _NB
            return lrow, col0

        def start_in(c, b):
            lrow, col0 = window(c)
            pltpu.async_copy(
                x_hbm.at[pl.ds(base + lrow, _NPAIR), pl.ds(col0, _NB)],
                bufs[b], isems[b])

        def wait_in(c, b):
            lrow, col0 = window(c)
            pltpu.make_async_copy(
                x_hbm.at[pl.ds(base + lrow, _NPAIR), pl.ds(col0, _NB)],
                bufs[b], isems[b]).wait()

        def start_out(c, b):
            lrow, col0 = window(c)
            pltpu.async_copy(
                bufs[b],
                out_hbm.at[pl.ds(base + lrow, _NPAIR), pl.ds(col0, _NB)],
                osems[b])

        def wait_out(c, b):
            lrow, col0 = window(c)
            pltpu.make_async_copy(
                bufs[b],
                out_hbm.at[pl.ds(base + lrow, _NPAIR), pl.ds(col0, _NB)],
                osems[b]).wait()

        def compute(c, b):
            lrow, _ = window(c)
            srow = lax.shift_right_logical(lrow, 3)
            sbase = lax.bitwise_and(lrow, 7) * _L
            splats = [
                pe_sv[srow, pl.ds(sbase + j * _L, _L)] for j in range(_NPAIR)
            ]

            @pl.loop(0, _NB // _L, unroll=8)
            def _vec(i):
                sl = pl.ds(i * _L, _L)
                for j in range(_NPAIR):
                    bufs[b][j, sl] = bufs[b][j, sl] + splats[j]

        for t in range(_PF):
            start_in(t, t % _RING)

        def step(c, cj):
            b = cj % _RING
            wait_in(c, b)
            compute(c, b)
            start_out(c, b)

        for c in head:
            step(c, c)
            t = c + _PF
            start_in(t, t % _RING)

        @pl.loop(main_lo, main_hi, step=_RING)
        def _main(ci):
            for j in range(_RING):
                c = ci + j
                cj = main_lo + j
                step(c, cj)
                tb = (cj + _PF) % _RING
                wait_out(c + _PF - _RING, tb)
                start_in(c + _PF, tb)

        for c in tail:
            step(c, c)
            t = c + _PF
            if t < n_chunks:
                tb = t % _RING
                wait_out(t - _RING, tb)
                start_in(t, tb)

        for t in range(n_chunks - _RING, n_chunks):
            wait_out(t, t % _RING)

    return _k


def kernel(x, pe_weight):
    B, S, D = x.shape
    P = S * D
    x2 = jnp.transpose(x, (1, 2, 0)).reshape(P, B)
    PW = 128
    pe2 = pe_weight[:S].reshape(P // PW, PW)
    out2 = _pe_add_kernel(P, B, PW)(x2, pe2)
    return jnp.transpose(out2.reshape(S, D, B), (2, 0, 1))


# speedup vs baseline: 1.2078x; 1.2078x over previous
import functools
import jax
import jax.numpy as jnp
from jax import lax
from jax.experimental import pallas as pl
from jax.experimental.pallas import tpu as pltpu
from jax.experimental.pallas import tpu_sc as plsc

_L = 16
_NPAIR = 2
_NB = 4096
_RING = 12
_PF = 6


def _pe_add_kernel(P, B, PW):
    info = plsc.get_sparse_core_info()
    NC, NS = info.num_cores, info.num_subcores
    NW = NC * NS
    assert P % (NW * _L) == 0 and B % _NB == 0
    rows_per_w = P // NW
    assert rows_per_w % _L == 0 and rows_per_w % _NPAIR == 0
    assert _NPAIR <= 8 and 8 % _NPAIR == 0
    n_halves = B // _NB
    n_chunks = (rows_per_w // _NPAIR) * n_halves
    mesh = plsc.VectorSubcoreMesh(core_axis_name="c", subcore_axis_name="s")

    head = list(range(_RING - _PF))
    main_lo = len(head)
    main_hi = main_lo + ((n_chunks - _PF - main_lo) // _RING) * _RING
    tail = list(range(main_hi, n_chunks))

    @functools.partial(
        pl.kernel,
        out_type=jax.ShapeDtypeStruct((P, B), jnp.float32),
        mesh=mesh,
        scratch_types=[
            pltpu.VMEM((P // PW, PW), jnp.float32),
            pltpu.VMEM((rows_per_w // 8, 8 * _L), jnp.float32),
            [pltpu.VMEM((_NPAIR, _NB), jnp.float32)] * _RING,
            [pltpu.SemaphoreType.DMA] * _RING,
            [pltpu.SemaphoreType.DMA] * _RING,
        ],
    )
    def _k(x_hbm, pe_hbm, out_hbm, pe_v, pe_sv, bufs, isems, osems):
        wid = lax.axis_index("s") * NC + lax.axis_index("c")
        pltpu.sync_copy(pe_hbm, pe_v)
        base = wid * rows_per_w

        @pl.loop(0, rows_per_w // _L)
        def _mk_splat(k):
            p0 = base + k * _L
            pev = pe_v[lax.shift_right_logical(p0, 7),
                       pl.ds(lax.bitwise_and(p0, PW - 1), _L)]
            k2 = 2 * k
            for j in range(_L):
                pe_sv[k2 + (j >> 3), pl.ds((j & 7) * _L, _L)] = (
                    lax.broadcast_in_dim(pev[j], (_L,), ()))

        def window(c):
            lrow = (c // n_halves) * _NPAIR
            col0 = (c % n_halves) * _NB
            return lrow, col0

        def start_in(c, b):
            lrow, col0 = window(c)
            pltpu.async_copy(
                x_hbm.at[pl.ds(base + lrow, _NPAIR), pl.ds(col0, _NB)],
                bufs[b], isems[b])

        def wait_in(c, b):
            lrow, col0 = window(c)
            pltpu.make_async_copy(
                x_hbm.at[pl.ds(base + lrow, _NPAIR), pl.ds(col0, _NB)],
                bufs[b], isems[b]).wait()

        def start_out(c, b):
            lrow, col0 = window(c)
            pltpu.async_copy(
                bufs[b],
                out_hbm.at[pl.ds(base + lrow, _NPAIR), pl.ds(col0, _NB)],
                osems[b])

        def wait_out(c, b):
            lrow, col0 = window(c)
            pltpu.make_async_copy(
                bufs[b],
                out_hbm.at[pl.ds(base + lrow, _NPAIR), pl.ds(col0, _NB)],
                osems[b]).wait()

        def compute(c, b):
            lrow, _ = window(c)
            srow = lax.shift_right_logical(lrow, 3)
            sbase = lax.bitwise_and(lrow, 7) * _L
            splats = [
                pe_sv[srow, pl.ds(sbase + j * _L, _L)] for j in range(_NPAIR)
            ]

            @pl.loop(0, _NB // _L, unroll=8)
            def _vec(i):
                sl = pl.ds(i * _L, _L)
                for j in range(_NPAIR):
                    bufs[b][j, sl] = bufs[b][j, sl] + splats[j]

        for t in range(_PF):
            start_in(t, t % _RING)

        def step(c, cj):
            b = cj % _RING
            wait_in(c, b)
            compute(c, b)
            start_out(c, b)

        for c in head:
            step(c, c)
            t = c + _PF
            start_in(t, t % _RING)

        @pl.loop(main_lo, main_hi, step=_RING)
        def _main(ci):
            for j in range(_RING):
                c = ci + j
                cj = main_lo + j
                step(c, cj)
                tb = (cj + _PF) % _RING
                wait_out(c + _PF - _RING, tb)
                start_in(c + _PF, tb)

        for c in tail:
            step(c, c)
            t = c + _PF
            if t < n_chunks:
                tb = t % _RING
                wait_out(t - _RING, tb)
                start_in(t, tb)

        for t in range(n_chunks - _RING, n_chunks):
            wait_out(t, t % _RING)

    return _k


def kernel(x, pe_weight):
    B, S, D = x.shape
    P = S * D
    x2 = jnp.transpose(x, (1, 2, 0)).reshape(P, B)
    PW = 128
    pe2 = pe_weight[:S].reshape(P // PW, PW)
    out2 = _pe_add_kernel(P, B, PW)(x2, pe2)
    return jnp.transpose(out2.reshape(S, D, B), (2, 0, 1))


# speedup vs baseline: 1.2134x; 1.0047x over previous
import functools
import jax
import jax.numpy as jnp
from jax import lax
from jax.experimental import pallas as pl
from jax.experimental.pallas import tpu as pltpu
from jax.experimental.pallas import tpu_sc as plsc

_L = 16
_NPAIR = 2
_NB = 4096
_RING = 12
_PF = 8


def _pe_add_kernel(P, B, PW):
    info = plsc.get_sparse_core_info()
    NC, NS = info.num_cores, info.num_subcores
    NW = NC * NS
    assert P % (NW * _L) == 0 and B % _NB == 0
    rows_per_w = P // NW
    assert rows_per_w % _L == 0 and rows_per_w % _NPAIR == 0
    assert _NPAIR <= 8 and 8 % _NPAIR == 0
    n_halves = B // _NB
    n_chunks = (rows_per_w // _NPAIR) * n_halves
    mesh = plsc.VectorSubcoreMesh(core_axis_name="c", subcore_axis_name="s")

    head = list(range(_RING - _PF))
    main_lo = len(head)
    main_hi = main_lo + ((n_chunks - _PF - main_lo) // _RING) * _RING
    tail = list(range(main_hi, n_chunks))

    @functools.partial(
        pl.kernel,
        out_type=jax.ShapeDtypeStruct((P, B), jnp.float32),
        mesh=mesh,
        scratch_types=[
            pltpu.VMEM((P // PW, PW), jnp.float32),
            pltpu.VMEM((rows_per_w // 8, 8 * _L), jnp.float32),
            [pltpu.VMEM((_NPAIR, _NB), jnp.float32)] * _RING,
            [pltpu.SemaphoreType.DMA] * _RING,
            [pltpu.SemaphoreType.DMA] * _RING,
        ],
    )
    def _k(x_hbm, pe_hbm, out_hbm, pe_v, pe_sv, bufs, isems, osems):
        wid = lax.axis_index("s") * NC + lax.axis_index("c")
        pltpu.sync_copy(pe_hbm, pe_v)
        base = wid * rows_per_w

        @pl.loop(0, rows_per_w // _L)
        def _mk_splat(k):
            p0 = base + k * _L
            pev = pe_v[lax.shift_right_logical(p0, 7),
                       pl.ds(lax.bitwise_and(p0, PW - 1), _L)]
            k2 = 2 * k
            for j in range(_L):
                pe_sv[k2 + (j >> 3), pl.ds((j & 7) * _L, _L)] = (
                    lax.broadcast_in_dim(pev[j], (_L,), ()))

        def window(c):
            lrow = (c // n_halves) * _NPAIR
            col0 = (c % n_halves) * _NB
            return lrow, col0

        def start_in(c, b):
            lrow, col0 = window(c)
            pltpu.async_copy(
                x_hbm.at[pl.ds(base + lrow, _NPAIR), pl.ds(col0, _NB)],
                bufs[b], isems[b])

        def wait_in(c, b):
            lrow, col0 = window(c)
            pltpu.make_async_copy(
                x_hbm.at[pl.ds(base + lrow, _NPAIR), pl.ds(col0, _NB)],
                bufs[b], isems[b]).wait()

        def start_out(c, b):
            lrow, col0 = window(c)
            pltpu.async_copy(
                bufs[b],
                out_hbm.at[pl.ds(base + lrow, _NPAIR), pl.ds(col0, _NB)],
                osems[b])

        def wait_out(c, b):
            lrow, col0 = window(c)
            pltpu.make_async_copy(
                bufs[b],
                out_hbm.at[pl.ds(base + lrow, _NPAIR), pl.ds(col0, _NB)],
                osems[b]).wait()

        def compute(c, b):
            lrow, _ = window(c)
            srow = lax.shift_right_logical(lrow, 3)
            sbase = lax.bitwise_and(lrow, 7) * _L
            splats = [
                pe_sv[srow, pl.ds(sbase + j * _L, _L)] for j in range(_NPAIR)
            ]

            @pl.loop(0, _NB // _L, unroll=8)
            def _vec(i):
                sl = pl.ds(i * _L, _L)
                for j in range(_NPAIR):
                    bufs[b][j, sl] = bufs[b][j, sl] + splats[j]

        for t in range(_PF):
            start_in(t, t % _RING)

        def step(c, cj):
            b = cj % _RING
            wait_in(c, b)
            compute(c, b)
            start_out(c, b)

        for c in head:
            step(c, c)
            t = c + _PF
            start_in(t, t % _RING)

        @pl.loop(main_lo, main_hi, step=_RING)
        def _main(ci):
            for j in range(_RING):
                c = ci + j
                cj = main_lo + j
                step(c, cj)
                tb = (cj + _PF) % _RING
                wait_out(c + _PF - _RING, tb)
                start_in(c + _PF, tb)

        for c in tail:
            step(c, c)
            t = c + _PF
            if t < n_chunks:
                tb = t % _RING
                wait_out(t - _RING, tb)
                start_in(t, tb)

        for t in range(n_chunks - _RING, n_chunks):
            wait_out(t, t % _RING)

    return _k


def kernel(x, pe_weight):
    B, S, D = x.shape
    P = S * D
    x2 = jnp.transpose(x, (1, 2, 0)).reshape(P, B)
    PW = 128
    pe2 = pe_weight[:S].reshape(P // PW, PW)
    out2 = _pe_add_kernel(P, B, PW)(x2, pe2)
    return jnp.transpose(out2.reshape(S, D, B), (2, 0, 1))


# speedup vs baseline: 1.2150x; 1.0013x over previous
import functools
import jax
import jax.numpy as jnp
from jax import lax
from jax.experimental import pallas as pl
from jax.experimental.pallas import tpu as pltpu
from jax.experimental.pallas import tpu_sc as plsc

_L = 16
_NPAIR = 2
_NB = 4096
_RING = 12
_PF = 10


def _pe_add_kernel(P, B, PW):
    info = plsc.get_sparse_core_info()
    NC, NS = info.num_cores, info.num_subcores
    NW = NC * NS
    assert P % (NW * _L) == 0 and B % _NB == 0
    rows_per_w = P // NW
    assert rows_per_w % _L == 0 and rows_per_w % _NPAIR == 0
    assert _NPAIR <= 8 and 8 % _NPAIR == 0
    n_halves = B // _NB
    n_chunks = (rows_per_w // _NPAIR) * n_halves
    mesh = plsc.VectorSubcoreMesh(core_axis_name="c", subcore_axis_name="s")

    head = list(range(_RING - _PF))
    main_lo = len(head)
    main_hi = main_lo + ((n_chunks - _PF - main_lo) // _RING) * _RING
    tail = list(range(main_hi, n_chunks))

    @functools.partial(
        pl.kernel,
        out_type=jax.ShapeDtypeStruct((P, B), jnp.float32),
        mesh=mesh,
        scratch_types=[
            pltpu.VMEM((P // PW, PW), jnp.float32),
            pltpu.VMEM((rows_per_w // 8, 8 * _L), jnp.float32),
            [pltpu.VMEM((_NPAIR, _NB), jnp.float32)] * _RING,
            [pltpu.SemaphoreType.DMA] * _RING,
            [pltpu.SemaphoreType.DMA] * _RING,
        ],
    )
    def _k(x_hbm, pe_hbm, out_hbm, pe_v, pe_sv, bufs, isems, osems):
        wid = lax.axis_index("s") * NC + lax.axis_index("c")
        pltpu.sync_copy(pe_hbm, pe_v)
        base = wid * rows_per_w

        @pl.loop(0, rows_per_w // _L)
        def _mk_splat(k):
            p0 = base + k * _L
            pev = pe_v[lax.shift_right_logical(p0, 7),
                       pl.ds(lax.bitwise_and(p0, PW - 1), _L)]
            k2 = 2 * k
            for j in range(_L):
                pe_sv[k2 + (j >> 3), pl.ds((j & 7) * _L, _L)] = (
                    lax.broadcast_in_dim(pev[j], (_L,), ()))

        def window(c):
            lrow = (c // n_halves) * _NPAIR
            col0 = (c % n_halves) * _NB
            return lrow, col0

        def start_in(c, b):
            lrow, col0 = window(c)
            pltpu.async_copy(
                x_hbm.at[pl.ds(base + lrow, _NPAIR), pl.ds(col0, _NB)],
                bufs[b], isems[b])

        def wait_in(c, b):
            lrow, col0 = window(c)
            pltpu.make_async_copy(
                x_hbm.at[pl.ds(base + lrow, _NPAIR), pl.ds(col0, _NB)],
                bufs[b], isems[b]).wait()

        def start_out(c, b):
            lrow, col0 = window(c)
            pltpu.async_copy(
                bufs[b],
                out_hbm.at[pl.ds(base + lrow, _NPAIR), pl.ds(col0, _NB)],
                osems[b])

        def wait_out(c, b):
            lrow, col0 = window(c)
            pltpu.make_async_copy(
                bufs[b],
                out_hbm.at[pl.ds(base + lrow, _NPAIR), pl.ds(col0, _NB)],
                osems[b]).wait()

        def compute(c, b):
            lrow, _ = window(c)
            srow = lax.shift_right_logical(lrow, 3)
            sbase = lax.bitwise_and(lrow, 7) * _L
            splats = [
                pe_sv[srow, pl.ds(sbase + j * _L, _L)] for j in range(_NPAIR)
            ]

            @pl.loop(0, _NB // _L, unroll=8)
            def _vec(i):
                sl = pl.ds(i * _L, _L)
                for j in range(_NPAIR):
                    bufs[b][j, sl] = bufs[b][j, sl] + splats[j]

        for t in range(_PF):
            start_in(t, t % _RING)

        def step(c, cj):
            b = cj % _RING
            wait_in(c, b)
            compute(c, b)
            start_out(c, b)

        for c in head:
            step(c, c)
            t = c + _PF
            start_in(t, t % _RING)

        @pl.loop(main_lo, main_hi, step=_RING)
        def _main(ci):
            for j in range(_RING):
                c = ci + j
                cj = main_lo + j
                step(c, cj)
                tb = (cj + _PF) % _RING
                wait_out(c + _PF - _RING, tb)
                start_in(c + _PF, tb)

        for c in tail:
            step(c, c)
            t = c + _PF
            if t < n_chunks:
                tb = t % _RING
                wait_out(t - _RING, tb)
                start_in(t, tb)

        for t in range(n_chunks - _RING, n_chunks):
            wait_out(t, t % _RING)

    return _k


def kernel(x, pe_weight):
    B, S, D = x.shape
    P = S * D
    x2 = jnp.transpose(x, (1, 2, 0)).reshape(P, B)
    PW = 128
    pe2 = pe_weight[:S].reshape(P // PW, PW)
    out2 = _pe_add_kernel(P, B, PW)(x2, pe2)
    return jnp.transpose(out2.reshape(S, D, B), (2, 0, 1))
